# baseline (speedup 1.0000x reference)
import functools

import jax
import jax.numpy as jnp
from jax import lax
from jax.experimental import pallas as pl
from jax.experimental.pallas import tpu as pltpu
from jax.experimental.pallas import tpu_sc as plsc

_N = 15_000_000
_B = 4096
_M = 4096

_GAMMA = 0.8
_TAU_MIN = 0.05
_TAU_MAX = 1.0
_RHO = 6.0
_ETA = 0.03
_BETA_U = 0.9
_GRAD_CLIP = 3.0

_NC = 2
_NS = 16
_NW = _NC * _NS
_BPW = _B // _NW

_CPW = 468_992
_PIECE = 28_672
_CP_FULL = 16
_CP_TAIL = _CPW - _CP_FULL * _PIECE
_CPLAST_TAIL = (_N - (_NW - 1) * _CPW) - _CP_FULL * _PIECE
_NSLOT = 4

_SC_SCRATCH = lambda: [
    pltpu.VMEM((_BPW,), jnp.int32),
    pltpu.VMEM((_BPW,), jnp.float32),
    pltpu.VMEM((_BPW,), jnp.float32),
    pltpu.VMEM((_BPW,), jnp.float32),
    pltpu.SemaphoreType.DMA,
    pltpu.SemaphoreType.DMA,
    pltpu.SemaphoreType.DMA,
]


@functools.cache
def _sc_kernels():
    mesh = plsc.VectorSubcoreMesh(core_axis_name="c", subcore_axis_name="s")

    @functools.partial(
        pl.kernel,
        out_type=[jax.ShapeDtypeStruct((_B,), jnp.float32)] * 3,
        mesh=mesh,
        scratch_types=_SC_SCRATCH(),
    )
    def _sc_gather(idx_hbm, s_hbm, t_hbm, u_hbm, so_hbm, to_hbm, uo_hbm,
                   idx_v, sv, tv, uv, sem0, sem1, sem2):
        wid = lax.axis_index("s") * _NC + lax.axis_index("c")
        base = wid * _BPW
        pltpu.sync_copy(idx_hbm.at[pl.ds(base, _BPW)], idx_v)
        c0 = pltpu.async_copy(s_hbm.at[idx_v], sv, sem0)
        c1 = pltpu.async_copy(t_hbm.at[idx_v], tv, sem1)
        c2 = pltpu.async_copy(u_hbm.at[idx_v], uv, sem2)
        c0.wait()
        c1.wait()
        c2.wait()
        pltpu.sync_copy(sv, so_hbm.at[pl.ds(base, _BPW)])
        pltpu.sync_copy(tv, to_hbm.at[pl.ds(base, _BPW)])
        pltpu.sync_copy(uv, uo_hbm.at[pl.ds(base, _BPW)])

    @functools.partial(
        pl.kernel,
        out_type=(),
        mesh=mesh,
        scratch_types=_SC_SCRATCH(),
    )
    def _sc_scatter(idx_hbm, sval_hbm, tval_hbm, uval_hbm, s_hbm, t_hbm, u_hbm,
                    idx_v, sv, tv, uv, sem0, sem1, sem2):
        wid = lax.axis_index("s") * _NC + lax.axis_index("c")
        base = wid * _BPW
        pltpu.sync_copy(idx_hbm.at[pl.ds(base, _BPW)], idx_v)
        pltpu.sync_copy(sval_hbm.at[pl.ds(base, _BPW)], sv)
        pltpu.sync_copy(tval_hbm.at[pl.ds(base, _BPW)], tv)
        pltpu.sync_copy(uval_hbm.at[pl.ds(base, _BPW)], uv)
        c0 = pltpu.async_copy(sv, s_hbm.at[idx_v], sem0)
        c1 = pltpu.async_copy(tv, t_hbm.at[idx_v], sem1)
        c2 = pltpu.async_copy(uv, u_hbm.at[idx_v], sem2)
        c0.wait()
        c1.wait()
        c2.wait()

    @functools.partial(
        pl.kernel,
        out_type=[jax.ShapeDtypeStruct((_N,), jnp.float32)] * 3,
        mesh=mesh,
        scratch_types=(
            [pltpu.VMEM((_PIECE,), jnp.float32)] * _NSLOT
            + [pltpu.SemaphoreType.DMA] * (2 * _NSLOT)
        ),
    )
    def _sc_copy(s_hbm, t_hbm, u_hbm, so_hbm, to_hbm, uo_hbm, *scratch):
        bufs = scratch[:_NSLOT]
        sems_in = scratch[_NSLOT:2 * _NSLOT]
        sems_out = scratch[2 * _NSLOT:]
        wid = lax.axis_index("s") * _NC + lax.axis_index("c")
        base = wid * _CPW

        def _copy_all(tail):
            pieces = []
            for a in range(3):
                for p in range(_CP_FULL):
                    pieces.append((a, p * _PIECE, _PIECE))
                pieces.append((a, _CP_FULL * _PIECE, tail))
            srcs = (s_hbm, t_hbm, u_hbm)
            dsts = (so_hbm, to_hbm, uo_hbm)

            def dma_in(p):
                a, off, sz = pieces[p]
                return pltpu.make_async_copy(
                    srcs[a].at[pl.ds(base + off, sz)],
                    bufs[p % _NSLOT].at[pl.ds(0, sz)], sems_in[p % _NSLOT])

            def dma_out(p):
                a, off, sz = pieces[p]
                return pltpu.make_async_copy(
                    bufs[p % _NSLOT].at[pl.ds(0, sz)],
                    dsts[a].at[pl.ds(base + off, sz)], sems_out[p % _NSLOT])

            n = len(pieces)
            for p in range(min(3, n)):
                dma_in(p).start()
            for p in range(n):
                dma_in(p).wait()
                dma_out(p).start()
                if p >= 1:
                    dma_out(p - 1).wait()
                if p + 3 < n:
                    dma_in(p + 3).start()
            dma_out(n - 1).wait()

        @pl.when(wid < _NW - 1)
        def _main():
            _copy_all(_CP_TAIL)

        @pl.when(wid == _NW - 1)
        def _last():
            _copy_all(_CPLAST_TAIL)

    return _sc_gather, _sc_scatter, _sc_copy


_GRID = 8
_RB = _B // _GRID


def _tc_compute_body(lg_ref, sg_ref, tg_ref, ug_ref,
                     sv_ref, uv_ref, tv_ref, loss_ref, gt_ref, ta_ref):
    i = pl.program_id(0)
    lg = lg_ref[...]
    pos = lg[:, 0:1]
    diff = lg - pos
    tau_b = tg_ref[...]
    inv_tau = 1.0 / tau_b
    dt = diff * inv_tau
    e = jnp.exp(dt)
    ones = jnp.ones((_M, 1), dtype=jnp.float32)
    s0 = jnp.dot(e, ones, preferred_element_type=jnp.float32)
    s1 = jnp.dot(e * diff, ones, preferred_element_type=jnp.float32)
    g = s0 * (1.0 / _M)
    s_new = (1.0 - _GAMMA) * sg_ref[...] + _GAMMA * g
    denom = _M * s_new
    loss_row = s1 / denom
    gt_row = jnp.log(s_new) + _RHO - loss_row * inv_tau
    gt_row = jnp.clip(gt_row, -_GRAD_CLIP, _GRAD_CLIP)
    u_new = (1.0 - _BETA_U) * ug_ref[...] + _BETA_U * gt_row
    t_new = jnp.clip(tau_b - _ETA * u_new, _TAU_MIN, _TAU_MAX)
    sv_ref[...] = s_new
    uv_ref[...] = u_new
    tv_ref[...] = t_new
    lp = jnp.sum(loss_row).reshape(1, 1)
    gp = jnp.sum(gt_row).reshape(1, 1)
    tp = jnp.sum(tau_b).reshape(1, 1)

    @pl.when(i == 0)
    def _init():
        loss_ref[...] = lp
        gt_ref[...] = gp
        ta_ref[...] = tp

    @pl.when(i > 0)
    def _acc():
        loss_ref[...] += lp
        gt_ref[...] += gp
        ta_ref[...] += tp


def _tc_compute(logits, s_g, tau_g, u_g):
    grid = _B // _RB
    col = pl.BlockSpec((_RB, 1), lambda i: (i, 0))
    scalar = pl.BlockSpec((1, 1), lambda i: (0, 0))
    return pl.pallas_call(
        _tc_compute_body,
        grid=(grid,),
        in_specs=[
            pl.BlockSpec((_RB, _M), lambda i: (i, 0)),
            col, col, col,
        ],
        out_specs=[col, col, col, scalar, scalar, scalar],
        out_shape=[jax.ShapeDtypeStruct((_B, 1), jnp.float32)] * 3
        + [jax.ShapeDtypeStruct((1, 1), jnp.float32)] * 3,
    )(logits, s_g.reshape(_B, 1), tau_g.reshape(_B, 1), u_g.reshape(_B, 1))


_CHUNK = 524_288

def _tc_copy_body(s_ref, so_ref):
    so_ref[...] = s_ref[...]


def _tc_copy(x):
    grid = pl.cdiv(_N, _CHUNK)
    spec = pl.BlockSpec((_CHUNK,), lambda i: (i,))
    return pl.pallas_call(
        _tc_copy_body,
        grid=(grid,),
        in_specs=[spec],
        out_specs=spec,
        out_shape=jax.ShapeDtypeStruct((_N,), jnp.float32),
    )(x)


def kernel(index, logits, s, tau, u):
    _sc_gather, _sc_scatter, _sc_copy = _sc_kernels()
    s_g, tau_g, u_g = _sc_gather(index, s, tau, u)
    s_c, tau_c, u_c = _sc_copy(s, tau, u)
    sv, uv, tv, loss_acc, gt_acc, ta_acc = _tc_compute(logits, s_g, tau_g, u_g)

    s_ref = jax.new_ref(s_c)
    tau_ref = jax.new_ref(tau_c)
    u_ref = jax.new_ref(u_c)
    _sc_scatter(index, sv.reshape(_B), tv.reshape(_B), uv.reshape(_B),
                s_ref, tau_ref, u_ref)
    s_new = s_ref[...]
    tau_new = tau_ref[...]
    u_new = u_ref[...]

    inv_b = jnp.float32(1.0 / _B)
    mean_loss = loss_acc[0, 0] * inv_b
    avg_tau = ta_acc[0, 0] * inv_b
    mean_gt = gt_acc[0, 0] * inv_b
    eta = jnp.float32(_ETA)
    return (mean_loss, avg_tau, eta, mean_gt, s_new, u_new, tau_new)


# speedup vs baseline: 1.0245x; 1.0245x over previous
import functools

import jax
import jax.numpy as jnp
from jax import lax
from jax.experimental import pallas as pl
from jax.experimental.pallas import tpu as pltpu
from jax.experimental.pallas import tpu_sc as plsc

_N = 15_000_000
_B = 4096
_M = 4096

_GAMMA = 0.8
_TAU_MIN = 0.05
_TAU_MAX = 1.0
_RHO = 6.0
_ETA = 0.03
_BETA_U = 0.9
_GRAD_CLIP = 3.0

_NC = 2
_NS = 16
_NW = _NC * _NS
_BPW = _B // _NW

_CPW = 468_992
_PIECE = 57_344
_CP_FULL = 8
_CP_TAIL = _CPW - _CP_FULL * _PIECE
_CPLAST_TAIL = (_N - (_NW - 1) * _CPW) - _CP_FULL * _PIECE

_SC_SCRATCH = lambda: [
    pltpu.VMEM((_BPW,), jnp.int32),
    pltpu.VMEM((_BPW,), jnp.float32),
    pltpu.VMEM((_BPW,), jnp.float32),
    pltpu.VMEM((_BPW,), jnp.float32),
    pltpu.SemaphoreType.DMA,
    pltpu.SemaphoreType.DMA,
    pltpu.SemaphoreType.DMA,
]


@functools.cache
def _sc_kernels():
    mesh = plsc.VectorSubcoreMesh(core_axis_name="c", subcore_axis_name="s")

    @functools.partial(
        pl.kernel,
        out_type=[jax.ShapeDtypeStruct((_B,), jnp.float32)] * 3,
        mesh=mesh,
        scratch_types=_SC_SCRATCH(),
    )
    def _sc_gather(idx_hbm, s_hbm, t_hbm, u_hbm, so_hbm, to_hbm, uo_hbm,
                   idx_v, sv, tv, uv, sem0, sem1, sem2):
        wid = lax.axis_index("s") * _NC + lax.axis_index("c")
        base = wid * _BPW
        pltpu.sync_copy(idx_hbm.at[pl.ds(base, _BPW)], idx_v)
        c0 = pltpu.async_copy(s_hbm.at[idx_v], sv, sem0)
        c1 = pltpu.async_copy(t_hbm.at[idx_v], tv, sem1)
        c2 = pltpu.async_copy(u_hbm.at[idx_v], uv, sem2)
        c0.wait()
        c1.wait()
        c2.wait()
        pltpu.sync_copy(sv, so_hbm.at[pl.ds(base, _BPW)])
        pltpu.sync_copy(tv, to_hbm.at[pl.ds(base, _BPW)])
        pltpu.sync_copy(uv, uo_hbm.at[pl.ds(base, _BPW)])

    @functools.partial(
        pl.kernel,
        out_type=(),
        mesh=mesh,
        scratch_types=_SC_SCRATCH() + [pltpu.SemaphoreType.DMA],
    )
    def _sc_scatter(idx_hbm, sval_hbm, tval_hbm, uval_hbm, s_hbm, t_hbm, u_hbm,
                    idx_v, sv, tv, uv, sem0, sem1, sem2, sem3):
        wid = lax.axis_index("s") * _NC + lax.axis_index("c")
        base = wid * _BPW
        l0 = pltpu.async_copy(idx_hbm.at[pl.ds(base, _BPW)], idx_v, sem0)
        l1 = pltpu.async_copy(sval_hbm.at[pl.ds(base, _BPW)], sv, sem1)
        l2 = pltpu.async_copy(tval_hbm.at[pl.ds(base, _BPW)], tv, sem2)
        l3 = pltpu.async_copy(uval_hbm.at[pl.ds(base, _BPW)], uv, sem3)
        l0.wait()
        l1.wait()
        l2.wait()
        l3.wait()
        c0 = pltpu.async_copy(sv, s_hbm.at[idx_v], sem0)
        c1 = pltpu.async_copy(tv, t_hbm.at[idx_v], sem1)
        c2 = pltpu.async_copy(uv, u_hbm.at[idx_v], sem2)
        c0.wait()
        c1.wait()
        c2.wait()

    @functools.partial(
        pl.kernel,
        out_type=[jax.ShapeDtypeStruct((_N,), jnp.float32)] * 2,
        mesh=mesh,
        scratch_types=[
            pltpu.VMEM((_PIECE,), jnp.float32),
            pltpu.VMEM((_PIECE,), jnp.float32),
            pltpu.SemaphoreType.DMA,
            pltpu.SemaphoreType.DMA,
            pltpu.SemaphoreType.DMA,
            pltpu.SemaphoreType.DMA,
        ],
    )
    def _sc_copy(t_hbm, u_hbm, to_hbm, uo_hbm,
                 buf0, buf1, sem_in0, sem_in1, sem_out0, sem_out1):
        wid = lax.axis_index("s") * _NC + lax.axis_index("c")
        base = wid * _CPW
        bufs = (buf0, buf1)
        sems_in = (sem_in0, sem_in1)
        sems_out = (sem_out0, sem_out1)

        def _copy_range(src, dst, tail):
            def dma_in(p, size):
                return pltpu.make_async_copy(
                    src.at[pl.ds(base + p * _PIECE, size)],
                    bufs[p % 2].at[pl.ds(0, size)], sems_in[p % 2])

            def dma_out(p, size):
                return pltpu.make_async_copy(
                    bufs[p % 2].at[pl.ds(0, size)],
                    dst.at[pl.ds(base + p * _PIECE, size)], sems_out[p % 2])

            sizes = [_PIECE] * _CP_FULL + [tail]
            last = len(sizes) - 1
            dma_in(0, sizes[0]).start()
            for p, sz in enumerate(sizes):
                dma_in(p, sz).wait()
                dma_out(p, sz).start()
                if p >= 1:
                    dma_out(p - 1, sizes[p - 1]).wait()
                if p + 1 <= last:
                    dma_in(p + 1, sizes[p + 1]).start()
            dma_out(last, sizes[last]).wait()

        for src, dst in ((t_hbm, to_hbm), (u_hbm, uo_hbm)):
            @pl.when(wid < _NW - 1)
            def _main(src=src, dst=dst):
                _copy_range(src, dst, _CP_TAIL)

            @pl.when(wid == _NW - 1)
            def _last(src=src, dst=dst):
                _copy_range(src, dst, _CPLAST_TAIL)

    return _sc_gather, _sc_scatter, _sc_copy


_GRID = 8
_RB = _B // _GRID


def _tc_compute_body(lg_ref, sg_ref, tg_ref, ug_ref,
                     sv_ref, uv_ref, tv_ref, loss_ref, gt_ref, ta_ref):
    i = pl.program_id(0)
    lg = lg_ref[...]
    pos = lg[:, 0:1]
    diff = lg - pos
    tau_b = tg_ref[...]
    inv_tau = 1.0 / tau_b
    dt = diff * inv_tau
    e = jnp.exp(dt)
    ones = jnp.ones((_M, 1), dtype=jnp.float32)
    s0 = jnp.dot(e, ones, preferred_element_type=jnp.float32)
    s1 = jnp.dot(e * diff, ones, preferred_element_type=jnp.float32)
    g = s0 * (1.0 / _M)
    s_new = (1.0 - _GAMMA) * sg_ref[...] + _GAMMA * g
    denom = _M * s_new
    loss_row = s1 / denom
    gt_row = jnp.log(s_new) + _RHO - loss_row * inv_tau
    gt_row = jnp.clip(gt_row, -_GRAD_CLIP, _GRAD_CLIP)
    u_new = (1.0 - _BETA_U) * ug_ref[...] + _BETA_U * gt_row
    t_new = jnp.clip(tau_b - _ETA * u_new, _TAU_MIN, _TAU_MAX)
    sv_ref[...] = s_new
    uv_ref[...] = u_new
    tv_ref[...] = t_new
    lp = jnp.sum(loss_row).reshape(1, 1)
    gp = jnp.sum(gt_row).reshape(1, 1)
    tp = jnp.sum(tau_b).reshape(1, 1)

    @pl.when(i == 0)
    def _init():
        loss_ref[...] = lp
        gt_ref[...] = gp
        ta_ref[...] = tp

    @pl.when(i > 0)
    def _acc():
        loss_ref[...] += lp
        gt_ref[...] += gp
        ta_ref[...] += tp


def _tc_compute(logits, s_g, tau_g, u_g):
    grid = _B // _RB
    col = pl.BlockSpec((_RB, 1), lambda i: (i, 0))
    scalar = pl.BlockSpec((1, 1), lambda i: (0, 0))
    return pl.pallas_call(
        _tc_compute_body,
        grid=(grid,),
        in_specs=[
            pl.BlockSpec((_RB, _M), lambda i: (i, 0)),
            col, col, col,
        ],
        out_specs=[col, col, col, scalar, scalar, scalar],
        out_shape=[jax.ShapeDtypeStruct((_B, 1), jnp.float32)] * 3
        + [jax.ShapeDtypeStruct((1, 1), jnp.float32)] * 3,
    )(logits, s_g.reshape(_B, 1), tau_g.reshape(_B, 1), u_g.reshape(_B, 1))


_CHUNK = 524_288

def _tc_copy_body(s_ref, so_ref):
    so_ref[...] = s_ref[...]


def _tc_copy(x):
    grid = pl.cdiv(_N, _CHUNK)
    spec = pl.BlockSpec((_CHUNK,), lambda i: (i,))
    return pl.pallas_call(
        _tc_copy_body,
        grid=(grid,),
        in_specs=[spec],
        out_specs=spec,
        out_shape=jax.ShapeDtypeStruct((_N,), jnp.float32),
    )(x)


def kernel(index, logits, s, tau, u):
    _sc_gather, _sc_scatter, _sc_copy = _sc_kernels()
    s_g, tau_g, u_g = _sc_gather(index, s, tau, u)
    tau_c, u_c = _sc_copy(tau, u)
    sv, uv, tv, loss_acc, gt_acc, ta_acc = _tc_compute(logits, s_g, tau_g, u_g)
    s_c = _tc_copy(s)

    s_ref = jax.new_ref(s_c)
    tau_ref = jax.new_ref(tau_c)
    u_ref = jax.new_ref(u_c)
    _sc_scatter(index, sv.reshape(_B), tv.reshape(_B), uv.reshape(_B),
                s_ref, tau_ref, u_ref)
    s_new = s_ref[...]
    tau_new = tau_ref[...]
    u_new = u_ref[...]

    inv_b = jnp.float32(1.0 / _B)
    mean_loss = loss_acc[0, 0] * inv_b
    avg_tau = ta_acc[0, 0] * inv_b
    mean_gt = gt_acc[0, 0] * inv_b
    eta = jnp.float32(_ETA)
    return (mean_loss, avg_tau, eta, mean_gt, s_new, u_new, tau_new)
